# baseline (speedup 1.0000x reference)
import functools

import jax
import jax.numpy as jnp
from jax import lax
from jax.experimental import pallas as pl
from jax.experimental.pallas import tpu as pltpu
from jax.experimental.pallas import tpu_sc as plsc


def _broadcast_sc(pos, batch_size):
    seq_len, embed_dim = pos.shape
    info = plsc.get_sparse_core_info()
    num_cores, num_subcores = info.num_cores, info.num_subcores
    num_workers = num_cores * num_subcores
    rows_per_worker = seq_len // num_workers
    chunk = min(rows_per_worker, 64)
    n_chunks = rows_per_worker // chunk

    mesh = plsc.VectorSubcoreMesh(core_axis_name="c", subcore_axis_name="s")

    @functools.partial(
        pl.kernel,
        mesh=mesh,
        out_type=jax.ShapeDtypeStruct((batch_size, seq_len, embed_dim), pos.dtype),
        scratch_types=[
            pltpu.VMEM((chunk, embed_dim), pos.dtype),
            pltpu.SemaphoreType.DMA,
        ],
    )
    def bcast(w_hbm, out_hbm, buf, wsem):
        wid = lax.axis_index("s") * num_cores + lax.axis_index("c")
        base = wid * rows_per_worker
        for c in range(n_chunks):
            r0 = base + c * chunk
            pltpu.sync_copy(w_hbm.at[pl.ds(r0, chunk)], buf)
            writes = [
                pltpu.async_copy(buf, out_hbm.at[b, pl.ds(r0, chunk)], wsem)
                for b in range(batch_size)
            ]
            for w in writes:
                w.wait()

    return bcast(pos)


def kernel(tokens, positional_embedding_weights):
    batch_size, seq_len = tokens.shape
    pos = positional_embedding_weights[:seq_len]
    return _broadcast_sc(pos, batch_size)
